# baseline (speedup 1.0000x reference)
import jax
import jax.numpy as jnp
from jax.experimental import pallas as pl
from jax.experimental.pallas import tpu as pltpu

_K = 1024
_N = 32000
_TM = 1024
_TN = 1280
_NJ = _N // _TN


def _matmul_bias_kernel(x_hbm, w_ref, b_ref, o_ref, x_vmem, sem):
    c = pl.program_id(0)
    j = pl.program_id(1)
    i = pl.program_id(2)
    ni = pl.num_programs(2)
    half = ni * _TM

    @pl.when(jnp.logical_and(j == 0, i == 0))
    def _():
        cp = pltpu.make_async_copy(
            x_hbm.at[pl.ds(c * half, half), :], x_vmem, sem
        )
        cp.start()
        cp.wait()

    x_bf = x_vmem[pl.ds(i * _TM, _TM), :].astype(jnp.bfloat16)
    w_bf = w_ref[...].astype(jnp.bfloat16)
    acc = jnp.dot(x_bf, w_bf, preferred_element_type=jnp.float32)
    o_ref[...] = acc + b_ref[...]


def kernel(x, w_kn, b):
    lead_shape = x.shape[:-1]
    x2d = x.reshape(-1, _K)
    M = x2d.shape[0]
    b_sl = b[:, :_N]

    ni_half = M // _TM // 2
    grid = (2, _NJ, ni_half)

    out = pl.pallas_call(
        _matmul_bias_kernel,
        out_shape=jax.ShapeDtypeStruct((M, _N), jnp.float32),
        grid=grid,
        in_specs=[
            pl.BlockSpec(memory_space=pl.ANY),
            pl.BlockSpec((_K, _TN), lambda c, j, i: (0, j)),
            pl.BlockSpec((1, _TN), lambda c, j, i: (0, j)),
        ],
        out_specs=pl.BlockSpec(
            (_TM, _TN), lambda c, j, i: (c * (M // _TM // 2) + i, j)
        ),
        scratch_shapes=[
            pltpu.VMEM((M // 2, _K), jnp.float32),
            pltpu.SemaphoreType.DMA,
        ],
        compiler_params=pltpu.CompilerParams(
            dimension_semantics=("parallel", "arbitrary", "arbitrary"),
            vmem_limit_bytes=64 * 1024 * 1024,
        ),
        cost_estimate=pl.CostEstimate(
            flops=2 * M * _N * _K,
            transcendentals=0,
            bytes_accessed=(
                x2d.size * 4 + _K * _N * 4 + b_sl.size * 4 + M * _N * 4
            ),
        ),
    )(x2d, w_kn, b_sl)

    return out.reshape(*lead_shape, _N)
